# baseline (speedup 1.0000x reference)
import jax
import jax.numpy as jnp
from jax import lax
from jax.experimental import pallas as pl
from jax.experimental.pallas import tpu as pltpu
from jax.experimental.pallas import tpu_sc as plsc

N_LAYERS = 4
K = 1024
D = 512
COMMIT_W = 0.25
TB = 2048
SB = 1024
N_TOK = 8192
NW = 32
T_PER_W = N_TOK // NW
CH = 128
NCH = T_PER_W // CH


def _dist_kernel(r_ref, cb_ref, xx_ref, idx_ref, cb2_ref, cc_ref):
    step = pl.program_id(0)

    @pl.when(step == 0)
    def _prep():
        cbv = cb_ref[...]
        cb2_ref[...] = cbv + cbv
        cc_ref[...] = jnp.broadcast_to(
            jnp.sum(cbv * cbv, axis=1)[None, :], cc_ref.shape)

    rb = r_ref[...]
    mm2 = lax.dot_general(rb, cb2_ref[...], (((1,), (1,)), ((), ())),
                          preferred_element_type=jnp.float32)
    d2 = (xx_ref[...] - mm2) + cc_ref[0:1, :]
    dist = jnp.sqrt(jnp.maximum(d2, 0.0))
    dmin = jnp.min(dist, axis=1, keepdims=True)
    ci1 = jax.lax.broadcasted_iota(jnp.int32, (1, K), 1).astype(jnp.float32)
    fidx = jnp.min(jnp.where(dist == dmin, ci1, jnp.float32(K)), axis=1,
                   keepdims=True)
    idx_ref[...] = fidx.astype(jnp.int32)


def _tc_dist(r, cb, xx):
    n = r.shape[0]
    return pl.pallas_call(
        _dist_kernel,
        grid=(n // TB,),
        in_specs=[
            pl.BlockSpec((TB, D), lambda t: (t, 0)),
            pl.BlockSpec((K, D), lambda t: (0, 0)),
            pl.BlockSpec((TB, 1), lambda t: (t, 0)),
        ],
        out_specs=pl.BlockSpec((TB, 1), lambda t: (t, 0)),
        out_shape=jax.ShapeDtypeStruct((n, 1), jnp.int32),
        scratch_shapes=[
            pltpu.VMEM((K, D), jnp.float32),
            pltpu.VMEM((8, K), jnp.float32),
        ],
    )(r, cb, xx)


def _sc_gather_kernel(cb_hbm, idx_hbm, q_hbm, idx_v, rows_v, sem):
    wid = lax.axis_index("s") * 2 + lax.axis_index("c")
    pltpu.sync_copy(idx_hbm.at[wid], idx_v)
    for c in range(NCH):
        pltpu.async_copy(cb_hbm.at[idx_v.at[c]], rows_v, sem).wait()
        base = wid * T_PER_W + c * CH
        pltpu.sync_copy(rows_v, q_hbm.at[pl.ds(base, CH)])


_SC_MESH = plsc.VectorSubcoreMesh(core_axis_name="c", subcore_axis_name="s")

_sc_gather = pl.kernel(
    _sc_gather_kernel, mesh=_SC_MESH,
    out_type=jax.ShapeDtypeStruct((N_TOK, D), jnp.float32),
    scratch_types=[
        pltpu.VMEM((NCH, CH), jnp.int32),
        pltpu.VMEM((CH, D), jnp.float32),
        pltpu.SemaphoreType.DMA,
    ],
)


def _sub_kernel(r_ref, q_ref, out_ref, loss_ref):
    step = pl.program_id(0)

    @pl.when(step == 0)
    def _init():
        loss_ref[...] = jnp.zeros_like(loss_ref)

    rn = r_ref[...] - q_ref[...]
    out_ref[...] = rn
    loss_ref[...] += jnp.full(loss_ref.shape, jnp.sum(rn * rn))


def _tc_sub(r, q):
    n = r.shape[0]
    return pl.pallas_call(
        _sub_kernel,
        grid=(n // SB,),
        in_specs=[
            pl.BlockSpec((SB, D), lambda t: (t, 0)),
            pl.BlockSpec((SB, D), lambda t: (t, 0)),
        ],
        out_specs=[
            pl.BlockSpec((SB, D), lambda t: (t, 0)),
            pl.BlockSpec((8, 128), lambda t: (0, 0)),
        ],
        out_shape=[
            jax.ShapeDtypeStruct((n, D), jnp.float32),
            jax.ShapeDtypeStruct((8, 128), jnp.float32),
        ],
    )(r, q)


def _sub_final_kernel(x_ref, r_ref, q_ref, quant_ref, rn_ref, loss_ref):
    step = pl.program_id(0)

    @pl.when(step == 0)
    def _init():
        loss_ref[...] = jnp.zeros_like(loss_ref)

    rn = r_ref[...] - q_ref[...]
    rn_ref[...] = rn
    quant_ref[...] = x_ref[...] - rn
    loss_ref[...] += jnp.full(loss_ref.shape, jnp.sum(rn * rn))


def _tc_sub_final(x, r, q):
    n = r.shape[0]
    return pl.pallas_call(
        _sub_final_kernel,
        grid=(n // SB,),
        in_specs=[
            pl.BlockSpec((SB, D), lambda t: (t, 0)),
            pl.BlockSpec((SB, D), lambda t: (t, 0)),
            pl.BlockSpec((SB, D), lambda t: (t, 0)),
        ],
        out_specs=[
            pl.BlockSpec((SB, D), lambda t: (t, 0)),
            pl.BlockSpec((SB, D), lambda t: (t, 0)),
            pl.BlockSpec((8, 128), lambda t: (0, 0)),
        ],
        out_shape=[
            jax.ShapeDtypeStruct((n, D), jnp.float32),
            jax.ShapeDtypeStruct((n, D), jnp.float32),
            jax.ShapeDtypeStruct((8, 128), jnp.float32),
        ],
    )(x, r, q)


def kernel(x, codebooks):
    b, s, d = x.shape
    x_flat = x.reshape(N_TOK, D)
    r = x_flat
    indices = []
    loss_parts = []
    for i in range(N_LAYERS):
        xx = jnp.sum(r * r, axis=1, keepdims=True)
        idx = _tc_dist(r, codebooks[i], xx)
        q = _sc_gather(codebooks[i], idx.reshape(NW, NCH, CH))
        if i < N_LAYERS - 1:
            r, lp = _tc_sub(r, q)
        else:
            quant_flat, r, lp = _tc_sub_final(x_flat, r, q)
        indices.append(idx.reshape(1, b, s))
        loss_parts.append(lp[0, 0])
    quantized = quant_flat.reshape(b, s, d)

    n_elem = jnp.float32(N_TOK * D)
    total_commit = jnp.float32(0.0)
    total_cb = jnp.float32(0.0)
    for i in range(N_LAYERS):
        m = loss_parts[i] / n_elem
        total_commit = total_commit + m * COMMIT_W
        total_cb = total_cb + m
    total_loss = total_commit + total_cb
    return (quantized, jnp.concatenate(indices, axis=0),
            total_commit, total_cb, total_loss)


# speedup vs baseline: 1.0173x; 1.0173x over previous
import jax
import jax.numpy as jnp
from jax import lax
from jax.experimental import pallas as pl
from jax.experimental.pallas import tpu as pltpu
from jax.experimental.pallas import tpu_sc as plsc

N_LAYERS = 4
K = 1024
D = 512
COMMIT_W = 0.25
TB = 1024
SB = 1024
N_TOK = 8192
NW = 32
T_PER_W = N_TOK // NW
CH = 128
NCH = T_PER_W // CH


def _dist_kernel(r_ref, cb_ref, xx_ref, idx_ref, cb2_ref, cc_ref):
    step = pl.program_id(0)

    @pl.when(step == 0)
    def _prep():
        cbv = cb_ref[...]
        cb2_ref[...] = cbv + cbv
        cc_ref[...] = jnp.broadcast_to(
            jnp.sum(cbv * cbv, axis=1)[None, :], cc_ref.shape)

    rb = r_ref[...]
    mm2 = lax.dot_general(rb, cb2_ref[...], (((1,), (1,)), ((), ())),
                          preferred_element_type=jnp.float32)
    d2 = (xx_ref[...] - mm2) + cc_ref[0:1, :]
    dist = jnp.sqrt(jnp.maximum(d2, 0.0))
    dmin = jnp.min(dist, axis=1, keepdims=True)
    ci1 = jax.lax.broadcasted_iota(jnp.int32, (1, K), 1).astype(jnp.float32)
    fidx = jnp.min(jnp.where(dist == dmin, ci1, jnp.float32(K)), axis=1,
                   keepdims=True)
    idx_ref[...] = fidx.astype(jnp.int32)


def _tc_dist(r, cb, xx):
    n = r.shape[0]
    return pl.pallas_call(
        _dist_kernel,
        grid=(n // TB,),
        in_specs=[
            pl.BlockSpec((TB, D), lambda t: (t, 0)),
            pl.BlockSpec((K, D), lambda t: (0, 0)),
            pl.BlockSpec((TB, 1), lambda t: (t, 0)),
        ],
        out_specs=pl.BlockSpec((TB, 1), lambda t: (t, 0)),
        out_shape=jax.ShapeDtypeStruct((n, 1), jnp.int32),
        scratch_shapes=[
            pltpu.VMEM((K, D), jnp.float32),
            pltpu.VMEM((8, K), jnp.float32),
        ],
    )(r, cb, xx)


def _sc_gather_kernel(cb_hbm, idx_hbm, q_hbm, idx_v, rows_v, sem):
    wid = lax.axis_index("s") * 2 + lax.axis_index("c")
    pltpu.sync_copy(idx_hbm.at[wid], idx_v)
    for c in range(NCH):
        pltpu.async_copy(cb_hbm.at[idx_v.at[c]], rows_v, sem).wait()
        base = wid * T_PER_W + c * CH
        pltpu.sync_copy(rows_v, q_hbm.at[pl.ds(base, CH)])


_SC_MESH = plsc.VectorSubcoreMesh(core_axis_name="c", subcore_axis_name="s")

_sc_gather = pl.kernel(
    _sc_gather_kernel, mesh=_SC_MESH,
    out_type=jax.ShapeDtypeStruct((N_TOK, D), jnp.float32),
    scratch_types=[
        pltpu.VMEM((NCH, CH), jnp.int32),
        pltpu.VMEM((CH, D), jnp.float32),
        pltpu.SemaphoreType.DMA,
    ],
)


def _sub_kernel(r_ref, q_ref, out_ref, loss_ref):
    step = pl.program_id(0)

    @pl.when(step == 0)
    def _init():
        loss_ref[...] = jnp.zeros_like(loss_ref)

    rn = r_ref[...] - q_ref[...]
    out_ref[...] = rn
    loss_ref[...] += jnp.full(loss_ref.shape, jnp.sum(rn * rn))


def _tc_sub(r, q):
    n = r.shape[0]
    return pl.pallas_call(
        _sub_kernel,
        grid=(n // SB,),
        in_specs=[
            pl.BlockSpec((SB, D), lambda t: (t, 0)),
            pl.BlockSpec((SB, D), lambda t: (t, 0)),
        ],
        out_specs=[
            pl.BlockSpec((SB, D), lambda t: (t, 0)),
            pl.BlockSpec((8, 128), lambda t: (0, 0)),
        ],
        out_shape=[
            jax.ShapeDtypeStruct((n, D), jnp.float32),
            jax.ShapeDtypeStruct((8, 128), jnp.float32),
        ],
    )(r, q)


def _sub_final_kernel(x_ref, r_ref, q_ref, quant_ref, rn_ref, loss_ref):
    step = pl.program_id(0)

    @pl.when(step == 0)
    def _init():
        loss_ref[...] = jnp.zeros_like(loss_ref)

    rn = r_ref[...] - q_ref[...]
    rn_ref[...] = rn
    quant_ref[...] = x_ref[...] - rn
    loss_ref[...] += jnp.full(loss_ref.shape, jnp.sum(rn * rn))


def _tc_sub_final(x, r, q):
    n = r.shape[0]
    return pl.pallas_call(
        _sub_final_kernel,
        grid=(n // SB,),
        in_specs=[
            pl.BlockSpec((SB, D), lambda t: (t, 0)),
            pl.BlockSpec((SB, D), lambda t: (t, 0)),
            pl.BlockSpec((SB, D), lambda t: (t, 0)),
        ],
        out_specs=[
            pl.BlockSpec((SB, D), lambda t: (t, 0)),
            pl.BlockSpec((SB, D), lambda t: (t, 0)),
            pl.BlockSpec((8, 128), lambda t: (0, 0)),
        ],
        out_shape=[
            jax.ShapeDtypeStruct((n, D), jnp.float32),
            jax.ShapeDtypeStruct((n, D), jnp.float32),
            jax.ShapeDtypeStruct((8, 128), jnp.float32),
        ],
    )(x, r, q)


def kernel(x, codebooks):
    b, s, d = x.shape
    x_flat = x.reshape(N_TOK, D)
    r = x_flat
    indices = []
    loss_parts = []
    for i in range(N_LAYERS):
        xx = jnp.sum(r * r, axis=1, keepdims=True)
        idx = _tc_dist(r, codebooks[i], xx)
        q = _sc_gather(codebooks[i], idx.reshape(NW, NCH, CH))
        if i < N_LAYERS - 1:
            r, lp = _tc_sub(r, q)
        else:
            quant_flat, r, lp = _tc_sub_final(x_flat, r, q)
        indices.append(idx.reshape(1, b, s))
        loss_parts.append(lp[0, 0])
    quantized = quant_flat.reshape(b, s, d)

    n_elem = jnp.float32(N_TOK * D)
    total_commit = jnp.float32(0.0)
    total_cb = jnp.float32(0.0)
    for i in range(N_LAYERS):
        m = loss_parts[i] / n_elem
        total_commit = total_commit + m * COMMIT_W
        total_cb = total_cb + m
    total_loss = total_commit + total_cb
    return (quantized, jnp.concatenate(indices, axis=0),
            total_commit, total_cb, total_loss)


# speedup vs baseline: 1.0270x; 1.0095x over previous
import jax
import jax.numpy as jnp
from jax import lax
from jax.experimental import pallas as pl
from jax.experimental.pallas import tpu as pltpu
from jax.experimental.pallas import tpu_sc as plsc

N_LAYERS = 4
K = 1024
D = 512
COMMIT_W = 0.25
TB = 1024
SB = 2048
N_TOK = 8192
NW = 32
T_PER_W = N_TOK // NW
CH = 128
NCH = T_PER_W // CH


def _dist_kernel(r_ref, cb_ref, xx_ref, idx_ref, cb2_ref, cc_ref):
    step = pl.program_id(0)

    @pl.when(step == 0)
    def _prep():
        cbv = cb_ref[...]
        cb2_ref[...] = cbv + cbv
        cc_ref[...] = jnp.broadcast_to(
            jnp.sum(cbv * cbv, axis=1)[None, :], cc_ref.shape)

    rb = r_ref[...]
    mm2 = lax.dot_general(rb, cb2_ref[...], (((1,), (1,)), ((), ())),
                          preferred_element_type=jnp.float32)
    d2 = (xx_ref[...] - mm2) + cc_ref[0:1, :]
    dist = jnp.sqrt(jnp.maximum(d2, 0.0))
    dmin = jnp.min(dist, axis=1, keepdims=True)
    ci1 = jax.lax.broadcasted_iota(jnp.int32, (1, K), 1).astype(jnp.float32)
    fidx = jnp.min(jnp.where(dist == dmin, ci1, jnp.float32(K)), axis=1,
                   keepdims=True)
    idx_ref[...] = fidx.astype(jnp.int32)


def _tc_dist(r, cb, xx):
    n = r.shape[0]
    return pl.pallas_call(
        _dist_kernel,
        grid=(n // TB,),
        in_specs=[
            pl.BlockSpec((TB, D), lambda t: (t, 0)),
            pl.BlockSpec((K, D), lambda t: (0, 0)),
            pl.BlockSpec((TB, 1), lambda t: (t, 0)),
        ],
        out_specs=pl.BlockSpec((TB, 1), lambda t: (t, 0)),
        out_shape=jax.ShapeDtypeStruct((n, 1), jnp.int32),
        scratch_shapes=[
            pltpu.VMEM((K, D), jnp.float32),
            pltpu.VMEM((8, K), jnp.float32),
        ],
    )(r, cb, xx)


def _sc_gather_kernel(cb_hbm, idx_hbm, q_hbm, idx_v, rows_v, sem):
    wid = lax.axis_index("s") * 2 + lax.axis_index("c")
    pltpu.sync_copy(idx_hbm.at[wid], idx_v)
    for c in range(NCH):
        pltpu.async_copy(cb_hbm.at[idx_v.at[c]], rows_v, sem).wait()
        base = wid * T_PER_W + c * CH
        pltpu.sync_copy(rows_v, q_hbm.at[pl.ds(base, CH)])


_SC_MESH = plsc.VectorSubcoreMesh(core_axis_name="c", subcore_axis_name="s")

_sc_gather = pl.kernel(
    _sc_gather_kernel, mesh=_SC_MESH,
    out_type=jax.ShapeDtypeStruct((N_TOK, D), jnp.float32),
    scratch_types=[
        pltpu.VMEM((NCH, CH), jnp.int32),
        pltpu.VMEM((CH, D), jnp.float32),
        pltpu.SemaphoreType.DMA,
    ],
)


def _sub_kernel(r_ref, q_ref, out_ref, loss_ref):
    step = pl.program_id(0)

    @pl.when(step == 0)
    def _init():
        loss_ref[...] = jnp.zeros_like(loss_ref)

    rn = r_ref[...] - q_ref[...]
    out_ref[...] = rn
    loss_ref[...] += jnp.full(loss_ref.shape, jnp.sum(rn * rn))


def _tc_sub(r, q):
    n = r.shape[0]
    return pl.pallas_call(
        _sub_kernel,
        grid=(n // SB,),
        in_specs=[
            pl.BlockSpec((SB, D), lambda t: (t, 0)),
            pl.BlockSpec((SB, D), lambda t: (t, 0)),
        ],
        out_specs=[
            pl.BlockSpec((SB, D), lambda t: (t, 0)),
            pl.BlockSpec((8, 128), lambda t: (0, 0)),
        ],
        out_shape=[
            jax.ShapeDtypeStruct((n, D), jnp.float32),
            jax.ShapeDtypeStruct((8, 128), jnp.float32),
        ],
    )(r, q)


def _sub_final_kernel(x_ref, r_ref, q_ref, quant_ref, rn_ref, loss_ref):
    step = pl.program_id(0)

    @pl.when(step == 0)
    def _init():
        loss_ref[...] = jnp.zeros_like(loss_ref)

    rn = r_ref[...] - q_ref[...]
    rn_ref[...] = rn
    quant_ref[...] = x_ref[...] - rn
    loss_ref[...] += jnp.full(loss_ref.shape, jnp.sum(rn * rn))


def _tc_sub_final(x, r, q):
    n = r.shape[0]
    return pl.pallas_call(
        _sub_final_kernel,
        grid=(n // SB,),
        in_specs=[
            pl.BlockSpec((SB, D), lambda t: (t, 0)),
            pl.BlockSpec((SB, D), lambda t: (t, 0)),
            pl.BlockSpec((SB, D), lambda t: (t, 0)),
        ],
        out_specs=[
            pl.BlockSpec((SB, D), lambda t: (t, 0)),
            pl.BlockSpec((SB, D), lambda t: (t, 0)),
            pl.BlockSpec((8, 128), lambda t: (0, 0)),
        ],
        out_shape=[
            jax.ShapeDtypeStruct((n, D), jnp.float32),
            jax.ShapeDtypeStruct((n, D), jnp.float32),
            jax.ShapeDtypeStruct((8, 128), jnp.float32),
        ],
    )(x, r, q)


def kernel(x, codebooks):
    b, s, d = x.shape
    x_flat = x.reshape(N_TOK, D)
    r = x_flat
    indices = []
    loss_parts = []
    for i in range(N_LAYERS):
        xx = jnp.sum(r * r, axis=1, keepdims=True)
        idx = _tc_dist(r, codebooks[i], xx)
        q = _sc_gather(codebooks[i], idx.reshape(NW, NCH, CH))
        if i < N_LAYERS - 1:
            r, lp = _tc_sub(r, q)
        else:
            quant_flat, r, lp = _tc_sub_final(x_flat, r, q)
        indices.append(idx.reshape(1, b, s))
        loss_parts.append(lp[0, 0])
    quantized = quant_flat.reshape(b, s, d)

    n_elem = jnp.float32(N_TOK * D)
    total_commit = jnp.float32(0.0)
    total_cb = jnp.float32(0.0)
    for i in range(N_LAYERS):
        m = loss_parts[i] / n_elem
        total_commit = total_commit + m * COMMIT_W
        total_cb = total_cb + m
    total_loss = total_commit + total_cb
    return (quantized, jnp.concatenate(indices, axis=0),
            total_commit, total_cb, total_loss)
